# baseline (speedup 1.0000x reference)
import functools

import jax
import jax.numpy as jnp
from jax import lax
from jax.experimental import pallas as pl
from jax.experimental.pallas import tpu as pltpu
from jax.experimental.pallas import tpu_sc as plsc

_DIM = 4096
_NE = 64
_K = 8
_TOKENS = 32768
_FLAT = _TOKENS * _K

_NC = 2
_NS = 16
_NW = _NC * _NS
_L = 16
_CHUNK = _FLAT // _NW
_PER_LANE = _CHUNK // _L

_TOK_BLOCK = 256


def _topk_body(x_ref, w_ref, scores_ref, idx_ref):
    x = x_ref[...]
    w = w_ref[...]
    s = lax.dot_general(x, w, (((1,), (1,)), ((), ())),
                        preferred_element_type=jnp.float32)
    m = jnp.max(s, axis=1, keepdims=True)
    e = jnp.exp(s - m)
    p = e / jnp.sum(e, axis=1, keepdims=True)
    iota = lax.broadcasted_iota(jnp.int32, p.shape, 1)
    vals = p
    scs = []
    ids = []
    for _ in range(_K):
        mk = jnp.max(vals, axis=1, keepdims=True)
        ik = jnp.min(jnp.where(vals == mk, iota, _NE), axis=1, keepdims=True)
        scs.append(mk)
        ids.append(ik)
        vals = jnp.where(iota == ik, -1.0, vals)
    scores_ref[...] = jnp.concatenate(scs, axis=1)
    idx_ref[...] = jnp.concatenate(ids, axis=1)


def _tc_topk(x, W):
    nblk = _TOKENS // _TOK_BLOCK
    return pl.pallas_call(
        _topk_body,
        grid=(nblk,),
        in_specs=[
            pl.BlockSpec((_TOK_BLOCK, _DIM), lambda i: (i, 0)),
            pl.BlockSpec((_NE, _DIM), lambda i: (0, 0)),
        ],
        out_specs=[
            pl.BlockSpec((_TOK_BLOCK, _K), lambda i: (i, 0)),
            pl.BlockSpec((_TOK_BLOCK, _K), lambda i: (i, 0)),
        ],
        out_shape=[
            jax.ShapeDtypeStruct((_TOKENS, _K), jnp.float32),
            jax.ShapeDtypeStruct((_TOKENS, _K), jnp.int32),
        ],
    )(x, W)


def _hist_body(keys_hbm, h_out, s_out, keys_v, hist_v, svec_v):
    c = lax.axis_index("c")
    s = lax.axis_index("s")
    w = s * _NC + c
    pltpu.sync_copy(keys_hbm.at[pl.ds(w * _CHUNK, _CHUNK)], keys_v)

    zeros16 = jnp.zeros((_L,), jnp.int32)
    for j in range(_NE * _L // _L):
        hist_v[pl.ds(j * _L, _L)] = zeros16

    lane = lax.broadcasted_iota(jnp.int32, (_L,), 0)
    base_idx = lane * _PER_LANE
    base_hist = lane * _NE
    ones = jnp.ones((_L,), jnp.int32)

    def body(t, carry):
        idx = base_idx + t
        k = plsc.load_gather(keys_v, [idx])
        plsc.addupdate_scatter(hist_v, [base_hist + k], ones)
        return carry

    lax.fori_loop(0, _PER_LANE, body, 0)

    for j in range(_NE // _L):
        acc = zeros16
        for l in range(_L):
            acc = acc + hist_v[pl.ds(l * _NE + j * _L, _L)]
        svec_v[pl.ds(j * _L, _L)] = acc

    pltpu.sync_copy(hist_v, h_out.at[pl.ds(w * _NE * _L, _NE * _L)])
    pltpu.sync_copy(svec_v, s_out.at[pl.ds(w * _NE, _NE)])


def _sc_hist(keys):
    mesh = plsc.VectorSubcoreMesh(core_axis_name="c", subcore_axis_name="s")
    f = pl.kernel(
        _hist_body,
        out_type=(
            jax.ShapeDtypeStruct((_NW * _NE * _L,), jnp.int32),
            jax.ShapeDtypeStruct((_NW * _NE,), jnp.int32),
        ),
        mesh=mesh,
        scratch_types=[
            pltpu.VMEM((_CHUNK,), jnp.int32),
            pltpu.VMEM((_NE * _L,), jnp.int32),
            pltpu.VMEM((_NE,), jnp.int32),
        ],
        compiler_params=pltpu.CompilerParams(needs_layout_passes=False),
    )
    return f(keys)


def _place_body(keys_hbm, scores_hbm, h_hbm, s_hbm,
                scores_out, tok_out, hist_out,
                keys_v, scores_v, outpos_v, tok_v, sall_v, h_v,
                counters_v, hist64_v, sem1, sem2):
    c = lax.axis_index("c")
    s = lax.axis_index("s")
    w = s * _NC + c
    pltpu.sync_copy(keys_hbm.at[pl.ds(w * _CHUNK, _CHUNK)], keys_v)
    pltpu.sync_copy(scores_hbm.at[pl.ds(w * _CHUNK, _CHUNK)], scores_v)
    pltpu.sync_copy(s_hbm, sall_v)
    pltpu.sync_copy(h_hbm.at[pl.ds(w * _NE * _L, _NE * _L)], h_v)

    zeros16 = jnp.zeros((_L,), jnp.int32)

    T = []
    C = []
    for j in range(_NE // _L):
        def acc_body(w2, carry, j=j):
            tj, cj = carry
            row = sall_v[pl.ds(w2 * _NE + j * _L, _L)]
            tj = tj + row
            cj = cj + jnp.where(w2 < w, row, zeros16)
            return (tj, cj)
        tj, cj = lax.fori_loop(0, _NW, acc_body, (zeros16, zeros16))
        T.append(tj)
        C.append(cj)

    G = []
    carry = jnp.zeros((), jnp.int32)
    for j in range(_NE // _L):
        cum = plsc.cumsum(T[j])
        G.append(cum - T[j] + carry)
        carry = carry + jnp.sum(T[j])

    @pl.when(w == 0)
    def _():
        for j in range(_NE // _L):
            hist64_v[pl.ds(j * _L, _L)] = T[j]
        pltpu.sync_copy(hist64_v, hist_out)

    acc = [G[j] + C[j] for j in range(_NE // _L)]
    for l in range(_L):
        for j in range(_NE // _L):
            counters_v[pl.ds(l * _NE + j * _L, _L)] = acc[j]
            acc[j] = acc[j] + h_v[pl.ds(l * _NE + j * _L, _L)]

    lane = lax.broadcasted_iota(jnp.int32, (_L,), 0)
    base_idx = lane * _PER_LANE
    base_hist = lane * _NE
    ones = jnp.ones((_L,), jnp.int32)
    gbase = w * _CHUNK

    def body(t, carry):
        idx = base_idx + t
        k = plsc.load_gather(keys_v, [idx])
        cidx = base_hist + k
        pos = plsc.load_gather(counters_v, [cidx])
        plsc.store_scatter(outpos_v, [idx], pos)
        plsc.addupdate_scatter(counters_v, [cidx], ones)
        tok = lax.shift_right_logical(gbase + idx, 3)
        plsc.store_scatter(tok_v, [idx], tok)
        return carry

    lax.fori_loop(0, _PER_LANE, body, 0)

    cp1 = pltpu.async_copy(scores_v, scores_out.at[outpos_v], sem1)
    cp2 = pltpu.async_copy(tok_v, tok_out.at[outpos_v], sem2)
    cp1.wait()
    cp2.wait()


def _sc_place(keys, scores_flat, h, s):
    mesh = plsc.VectorSubcoreMesh(core_axis_name="c", subcore_axis_name="s")
    f = pl.kernel(
        _place_body,
        out_type=(
            jax.ShapeDtypeStruct((_FLAT,), jnp.float32),
            jax.ShapeDtypeStruct((_FLAT,), jnp.int32),
            jax.ShapeDtypeStruct((_NE,), jnp.int32),
        ),
        mesh=mesh,
        scratch_types=[
            pltpu.VMEM((_CHUNK,), jnp.int32),
            pltpu.VMEM((_CHUNK,), jnp.float32),
            pltpu.VMEM((_CHUNK,), jnp.int32),
            pltpu.VMEM((_CHUNK,), jnp.int32),
            pltpu.VMEM((_NW * _NE,), jnp.int32),
            pltpu.VMEM((_NE * _L,), jnp.int32),
            pltpu.VMEM((_NE * _L,), jnp.int32),
            pltpu.VMEM((_NE,), jnp.int32),
            pltpu.SemaphoreType.DMA,
            pltpu.SemaphoreType.DMA,
        ],
        compiler_params=pltpu.CompilerParams(needs_layout_passes=False),
    )
    return f(keys, scores_flat, h, s)


def kernel(x, W):
    top_scores, sel = _tc_topk(x, W)
    keys = sel.reshape(-1)
    scores_flat = top_scores.reshape(-1)
    h, s = _sc_hist(keys)
    out_scores, out_tok, hist = _sc_place(keys, scores_flat, h, s)
    return out_scores, out_tok, hist
